# Initial kernel scaffold; baseline (speedup 1.0000x reference)
import functools

import jax
import jax.numpy as jnp
from jax import lax
from jax.experimental import pallas as pl
from jax.experimental.pallas import tpu as pltpu
from jax.experimental.pallas import tpu_sc as plsc

F32 = jnp.float32
N_NODES = 10000
N_EDGES = 320000
N_GRAPHS = 64
H = 128
EP = N_EDGES // 128
NC, NS, NW = 2, 16, 32
ME = 2000
MN = 2000

_MESH = plsc.VectorSubcoreMesh(core_axis_name="c", subcore_axis_name="s")


def _pad2(a, r, c):
    return jnp.pad(a, ((0, r - a.shape[0]), (0, c - a.shape[1])))


def _pad_mlp(p, n_in_pad, ln):
    Ws, bs = p["W"], p["b"]
    W0 = _pad2(Ws[0], n_in_pad, H)
    W1 = _pad2(Ws[1], H, H)
    W2 = _pad2(Ws[2], H, H)
    W3 = _pad2(Ws[3], H, H)
    b0 = jnp.pad(bs[0], (0, H - bs[0].shape[0]))[None, :]
    b1 = jnp.pad(bs[1], (0, H - bs[1].shape[0]))[None, :]
    b2 = jnp.pad(bs[2], (0, H - bs[2].shape[0]))[None, :]
    b3 = jnp.pad(bs[3], (0, H - bs[3].shape[0]))[None, :]
    out = [W0, b0, W1, b1, W2, b2, W3, b3]
    if ln:
        out += [jnp.pad(p["g"], (0, H - p["g"].shape[0]))[None, :],
                jnp.pad(p["be"], (0, H - p["be"].shape[0]))[None, :]]
    return out


def _mlp_tail(t, W1, b1, W2, b2, W3, b3, g=None, be=None):
    t = jnp.maximum(t, 0.0)
    t = jnp.maximum(jnp.dot(t, W1, preferred_element_type=F32) + b1, 0.0)
    t = jnp.maximum(jnp.dot(t, W2, preferred_element_type=F32) + b2, 0.0)
    t = jnp.dot(t, W3, preferred_element_type=F32) + b3
    if g is not None:
        mu = jnp.mean(t, axis=-1, keepdims=True)
        var = jnp.mean((t - mu) ** 2, axis=-1, keepdims=True)
        t = (t - mu) * lax.rsqrt(var + 1e-5) * g + be
    return t


def _full(shape):
    return pl.BlockSpec(shape, lambda *_: tuple(0 for _ in shape))


@functools.lru_cache(maxsize=None)
def _make_sc_gather(D):
    per = EP // NW

    def body(tab, rp, cp, hr, hc, idx, bufr, bufc, sem1, sem2):
        cid = lax.axis_index("c")
        sid = lax.axis_index("s")
        wid = sid * NC + cid

        def step(t, carry):
            j = t * NW + wid

            @pl.when(j < EP)
            def _():
                pltpu.sync_copy(rp.at[j], idx.at[0])
                pltpu.sync_copy(cp.at[j], idx.at[1])
                cr = pltpu.async_copy(tab.at[idx.at[0]], bufr, sem1)
                cc = pltpu.async_copy(tab.at[idx.at[1]], bufc, sem2)
                cr.wait()
                cc.wait()
                pltpu.sync_copy(bufr, hr.at[pl.ds(j * 128, 128)])
                pltpu.sync_copy(bufc, hc.at[pl.ds(j * 128, 128)])

            return carry

        lax.fori_loop(0, per + 1, step, 0)

    fs = jax.ShapeDtypeStruct((N_EDGES, D), F32)
    return pl.kernel(
        body,
        out_type=[fs, fs],
        mesh=_MESH,
        scratch_types=[
            pltpu.VMEM((2, 128), jnp.int32),
            pltpu.VMEM((128, D), F32),
            pltpu.VMEM((128, D), F32),
            pltpu.SemaphoreType.DMA,
            pltpu.SemaphoreType.DMA,
        ],
    )


def _sc_gather(table, rowp, colp):
    return _make_sc_gather(table.shape[1])(table, rowp, colp)


@functools.lru_cache(maxsize=None)
def _make_sc_scatter():
    half = EP // NC
    stripe = N_NODES // NS

    def body(mh, cph, out, idx, mbuf, acc, sem):
        cid = lax.axis_index("c")
        sid = lax.axis_index("s")

        def zb(i, carry):
            r = i // 8
            c = (i % 8) * 16
            mbuf[r, pl.ds(c, 16)] = jnp.zeros((16,), F32)
            return carry

        lax.fori_loop(0, 1024, zb, 0)

        def zs(k, carry):
            pltpu.sync_copy(mbuf.at[pl.ds(0, 125)],
                            acc.at[pl.ds(sid * stripe + k * 125, 125)])
            return carry

        lax.fori_loop(0, 5, zs, 0)
        plsc.subcore_barrier()

        def step(t, carry):
            jl = t * NS + sid

            @pl.when(jl < half)
            def _():
                j = cid * half + jl
                pltpu.sync_copy(cph.at[j], idx.at[0])
                pltpu.sync_copy(mh.at[pl.ds(j * 128, 128)], mbuf)
                pltpu.sync_copy(mbuf, acc.at[idx.at[0]], add=True)

            return carry

        lax.fori_loop(0, half // NS + 1, step, 0)
        plsc.subcore_barrier()

        def co(k, carry):
            r0 = sid * stripe + k * 125
            pltpu.sync_copy(acc.at[pl.ds(r0, 125)], mbuf.at[pl.ds(0, 125)])
            pltpu.sync_copy(mbuf.at[pl.ds(0, 125)], out.at[cid, pl.ds(r0, 125)])
            return carry

        lax.fori_loop(0, 5, co, 0)

    return pl.kernel(
        body,
        out_type=jax.ShapeDtypeStruct((NC, N_NODES, H), F32),
        mesh=_MESH,
        scratch_types=[
            pltpu.VMEM((1, 128), jnp.int32),
            pltpu.VMEM((128, H), F32),
            pltpu.VMEM_SHARED((N_NODES, H), F32),
            pltpu.SemaphoreType.DMA,
        ],
    )


def _sc_scatter(m, colp):
    return _make_sc_scatter()(m, colp)


def _node_enc_body(x, W0, b0, W1, b1, W2, b2, W3, b3, g, be, ho):
    t = jnp.dot(x[...], W0[...], preferred_element_type=F32) + b0[...]
    ho[...] = _mlp_tail(t, W1[...], b1[...], W2[...], b2[...], W3[...], b3[...],
                        g[...], be[...])


def _tc_node_enc(x, w):
    grid = N_NODES // MN
    specs = [pl.BlockSpec((MN, 4), lambda i: (i, 0))]
    specs += [_full(a.shape) for a in w]
    return pl.pallas_call(
        _node_enc_body,
        grid=(grid,),
        in_specs=specs,
        out_specs=pl.BlockSpec((MN, H), lambda i: (i, 0)),
        out_shape=jax.ShapeDtypeStruct((N_NODES, H), F32),
    )(x, *w)


def _edge_enc_body(pr, pc, W0, b0, W1, b1, W2, b2, W3, b3, g, be, eo):
    d = pr[...] - pc[...]
    t = jnp.dot(d, W0[...], preferred_element_type=F32) + b0[...]
    eo[...] = _mlp_tail(t, W1[...], b1[...], W2[...], b2[...], W3[...], b3[...],
                        g[...], be[...])


def _tc_edge_enc(prow, pcol, w):
    grid = N_EDGES // ME
    specs = [pl.BlockSpec((ME, 16), lambda i: (i, 0)),
             pl.BlockSpec((ME, 16), lambda i: (i, 0))]
    specs += [_full(a.shape) for a in w]
    return pl.pallas_call(
        _edge_enc_body,
        grid=(grid,),
        in_specs=specs,
        out_specs=pl.BlockSpec((ME, H), lambda i: (i, 0)),
        out_shape=jax.ShapeDtypeStruct((N_EDGES, H), F32),
    )(prow, pcol, *w)


def _edge_block_body(hr, hc, e, pr, u,
                     W0, b0, W1, b1, W2, b2, W3, b3, g, be,
                     V0, c0, V1, c1, V2, c2, V3, c3, vg, vbe,
                     eo, mo):
    hr_ = hr[...]
    hc_ = hc[...]
    e_ = e[...]
    bef = pr[...][:, 3]
    oh = (bef[:, None] == lax.broadcasted_iota(F32, (ME, N_GRAPHS), 1)
          ).astype(F32)
    ube = jnp.dot(oh, u[...], preferred_element_type=F32)
    W0_ = W0[...]
    t = (jnp.dot(hr_, W0_[0:128], preferred_element_type=F32)
         + jnp.dot(hc_, W0_[128:256], preferred_element_type=F32)
         + jnp.dot(e_, W0_[256:384], preferred_element_type=F32)
         + jnp.dot(ube, W0_[384:512], preferred_element_type=F32)
         + b0[...])
    de = _mlp_tail(t, W1[...], b1[...], W2[...], b2[...], W3[...], b3[...],
                   g[...], be[...])
    e_new = e_ + de
    eo[...] = e_new
    V0_ = V0[...]
    s = (jnp.dot(hr_, V0_[0:128], preferred_element_type=F32)
         + jnp.dot(e_new, V0_[128:256], preferred_element_type=F32)
         + c0[...])
    mo[...] = _mlp_tail(s, V1[...], c1[...], V2[...], c2[...], V3[...], c3[...],
                        vg[...], vbe[...])


def _tc_edge_block(hr, hc, e, prow, u, we, wv):
    grid = N_EDGES // ME
    em = lambda i: (i, 0)
    specs = [pl.BlockSpec((ME, H), em), pl.BlockSpec((ME, H), em),
             pl.BlockSpec((ME, H), em), pl.BlockSpec((ME, 16), em),
             _full((N_GRAPHS, H))]
    specs += [_full(a.shape) for a in we]
    specs += [_full(a.shape) for a in wv]
    return pl.pallas_call(
        _edge_block_body,
        grid=(grid,),
        in_specs=specs,
        out_specs=[pl.BlockSpec((ME, H), em), pl.BlockSpec((ME, H), em)],
        out_shape=[jax.ShapeDtypeStruct((N_EDGES, H), F32),
                   jax.ShapeDtypeStruct((N_EDGES, H), F32)],
    )(hr, hc, e, prow, u, *we, *wv)


def _node_block_body(h, a0, a1, T, u,
                     U0, d0, U1, d1, U2, d2, U3, d3, ug, ube,
                     Q0, q0, Q1, q1, Q2, q2, Q3, q3, qg, qbe,
                     ho, uo, gacc):
    i = pl.program_id(0)
    h_ = h[...]
    agg = a0[...] + a1[...]
    bf = T[...][:, 3]
    ohb = (bf[:, None] == lax.broadcasted_iota(F32, (MN, N_GRAPHS), 1)
           ).astype(F32)
    u_ = u[...]
    ub = jnp.dot(ohb, u_, preferred_element_type=F32)
    U0_ = U0[...]
    t = (jnp.dot(h_, U0_[0:128], preferred_element_type=F32)
         + jnp.dot(agg, U0_[128:256], preferred_element_type=F32)
         + jnp.dot(ub, U0_[256:384], preferred_element_type=F32)
         + d0[...])
    hn = h_ + _mlp_tail(t, U1[...], d1[...], U2[...], d2[...], U3[...], d3[...],
                        ug[...], ube[...])
    ho[...] = hn
    part = lax.dot_general(ohb, hn, (((0,), (0,)), ((), ())),
                           preferred_element_type=F32)

    @pl.when(i == 0)
    def _():
        gacc[...] = part

    @pl.when(i > 0)
    def _():
        gacc[...] = gacc[...] + part

    @pl.when(i == pl.num_programs(0) - 1)
    def _():
        Q0_ = Q0[...]
        tq = (jnp.dot(u_, Q0_[0:128], preferred_element_type=F32)
              + jnp.dot(gacc[...], Q0_[128:256], preferred_element_type=F32)
              + q0[...])
        uo[...] = u_ + _mlp_tail(tq, Q1[...], q1[...], Q2[...], q2[...],
                                 Q3[...], q3[...], qg[...], qbe[...])


def _tc_node_block(h, a0, a1, T, u, wu, wq):
    grid = N_NODES // MN
    em = lambda i: (i, 0)
    specs = [pl.BlockSpec((MN, H), em), pl.BlockSpec((MN, H), em),
             pl.BlockSpec((MN, H), em), pl.BlockSpec((MN, 16), em),
             _full((N_GRAPHS, H))]
    specs += [_full(a.shape) for a in wu]
    specs += [_full(a.shape) for a in wq]
    return pl.pallas_call(
        _node_block_body,
        grid=(grid,),
        in_specs=specs,
        out_specs=[pl.BlockSpec((MN, H), em), _full((N_GRAPHS, H))],
        out_shape=[jax.ShapeDtypeStruct((N_NODES, H), F32),
                   jax.ShapeDtypeStruct((N_GRAPHS, H), F32)],
        scratch_shapes=[pltpu.VMEM((N_GRAPHS, H), F32)],
    )(h, a0, a1, T, u, *wu, *wq)


def _decoder_body(h, W0, b0, W1, b1, W2, b2, W3, b3, yo):
    t = jnp.dot(h[...], W0[...], preferred_element_type=F32) + b0[...]
    yo[...] = _mlp_tail(t, W1[...], b1[...], W2[...], b2[...], W3[...], b3[...])


def _tc_decoder(h, w):
    grid = N_NODES // MN
    specs = [pl.BlockSpec((MN, H), lambda i: (i, 0))]
    specs += [_full(a.shape) for a in w]
    return pl.pallas_call(
        _decoder_body,
        grid=(grid,),
        in_specs=specs,
        out_specs=pl.BlockSpec((MN, H), lambda i: (i, 0)),
        out_shape=jax.ShapeDtypeStruct((N_NODES, H), F32),
    )(h, *w)


def kernel(x, edge_index, batch, params):
    rowp = edge_index[0].astype(jnp.int32).reshape(EP, 128)
    colp = edge_index[1].astype(jnp.int32).reshape(EP, 128)
    batchf = batch.astype(F32)
    T = jnp.concatenate(
        [x[:, :3], batchf[:, None], jnp.zeros((N_NODES, 12), F32)], axis=1)

    w_nenc = _pad_mlp(params["node_enc"], 4, True)
    w_nenc[0] = jnp.zeros((4, H), F32).at[3, :].set(w_nenc[0][0])
    w_eenc = _pad_mlp(params["edge_enc"], 16, True)
    w_dec = _pad_mlp(params["decoder"], H, False)

    prow, pcol = _sc_gather(T, rowp, colp)
    h = _tc_node_enc(x, w_nenc)
    e = _tc_edge_enc(prow, pcol, w_eenc)
    u = jnp.zeros((N_GRAPHS, H), F32)

    for bp in params["blocks"]:
        we = _pad_mlp(bp["edge"], 4 * H, True)
        wv = _pad_mlp(bp["node1"], 2 * H, True)
        wu = _pad_mlp(bp["node2"], 3 * H, True)
        wq = _pad_mlp(bp["glob"], 2 * H, True)
        hr, hc = _sc_gather(h, rowp, colp)
        e, m = _tc_edge_block(hr, hc, e, prow, u, we, wv)
        parts = _sc_scatter(m, colp)
        h, u = _tc_node_block(h, parts[0], parts[1], T, u, wu, wq)

    y = _tc_decoder(h, w_dec)
    return y[:, :1]


# speedup vs baseline: 3.4485x; 3.4485x over previous
import functools

import jax
import jax.numpy as jnp
from jax import lax
from jax.experimental import pallas as pl
from jax.experimental.pallas import tpu as pltpu
from jax.experimental.pallas import tpu_sc as plsc

F32 = jnp.float32
N_NODES = 10000
N_EDGES = 320000
N_GRAPHS = 64
H = 128
EP = N_EDGES // 128
NC, NS, NW = 2, 16, 32
ME = 2000
MN = 2000

@functools.lru_cache(maxsize=None)
def _mesh():
    return plsc.VectorSubcoreMesh(core_axis_name="c", subcore_axis_name="s",
                                  num_cores=NC, num_subcores=NS)


def _pad2(a, r, c):
    return jnp.pad(a, ((0, r - a.shape[0]), (0, c - a.shape[1])))


def _pad_mlp(p, n_in_pad, ln):
    Ws, bs = p["W"], p["b"]
    W0 = _pad2(Ws[0], n_in_pad, H)
    W1 = _pad2(Ws[1], H, H)
    W2 = _pad2(Ws[2], H, H)
    W3 = _pad2(Ws[3], H, H)
    b0 = jnp.pad(bs[0], (0, H - bs[0].shape[0]))[None, :]
    b1 = jnp.pad(bs[1], (0, H - bs[1].shape[0]))[None, :]
    b2 = jnp.pad(bs[2], (0, H - bs[2].shape[0]))[None, :]
    b3 = jnp.pad(bs[3], (0, H - bs[3].shape[0]))[None, :]
    out = [W0, b0, W1, b1, W2, b2, W3, b3]
    if ln:
        out += [jnp.pad(p["g"], (0, H - p["g"].shape[0]))[None, :],
                jnp.pad(p["be"], (0, H - p["be"].shape[0]))[None, :]]
    return out


def _mlp_tail(t, W1, b1, W2, b2, W3, b3, g=None, be=None):
    t = jnp.maximum(t, 0.0)
    t = jnp.maximum(jnp.dot(t, W1, preferred_element_type=F32) + b1, 0.0)
    t = jnp.maximum(jnp.dot(t, W2, preferred_element_type=F32) + b2, 0.0)
    t = jnp.dot(t, W3, preferred_element_type=F32) + b3
    if g is not None:
        mu = jnp.mean(t, axis=-1, keepdims=True)
        var = jnp.mean((t - mu) ** 2, axis=-1, keepdims=True)
        t = (t - mu) * lax.rsqrt(var + 1e-5) * g + be
    return t


def _full(shape):
    return pl.BlockSpec(shape, lambda *_: tuple(0 for _ in shape))


@functools.lru_cache(maxsize=None)
def _make_sc_gather(D):
    per = EP // NW

    def body(tab, rp, cp, hr, hc, idx, bufr, bufc, sem1, sem2):
        cid = lax.axis_index("c")
        sid = lax.axis_index("s")
        wid = sid * NC + cid

        def step(t, carry):
            j = t * NW + wid

            @pl.when(j < EP)
            def _():
                pltpu.sync_copy(rp.at[j], idx.at[0])
                pltpu.sync_copy(cp.at[j], idx.at[1])
                cr = pltpu.async_copy(tab.at[idx.at[0]], bufr, sem1)
                cc = pltpu.async_copy(tab.at[idx.at[1]], bufc, sem2)
                cr.wait()
                cc.wait()
                pltpu.sync_copy(bufr, hr.at[pl.ds(j * 128, 128)])
                pltpu.sync_copy(bufc, hc.at[pl.ds(j * 128, 128)])

            return carry

        lax.fori_loop(0, per + 1, step, 0)

    fs = jax.ShapeDtypeStruct((N_EDGES, D), F32)
    return pl.kernel(
        body,
        out_type=[fs, fs],
        mesh=_mesh(),
        compiler_params=pltpu.CompilerParams(use_tc_tiling_on_sc=(D == 128)),
        scratch_types=[
            pltpu.VMEM((2, 128), jnp.int32),
            pltpu.VMEM((128, D), F32),
            pltpu.VMEM((128, D), F32),
            pltpu.SemaphoreType.DMA,
            pltpu.SemaphoreType.DMA,
        ],
    )


def _sc_gather(table, rowp, colp):
    return _make_sc_gather(table.shape[1])(table, rowp, colp)


@functools.lru_cache(maxsize=None)
def _make_sc_scatter():
    half = EP // NC
    nch = N_NODES // 128
    tail = N_NODES - nch * 128

    def body(mh, cph, out, idx, mbuf, acc, sem):
        cid = lax.axis_index("c")
        sid = lax.axis_index("s")

        def zb(i, carry):
            r = i // 8
            c = (i % 8) * 16
            mbuf[r, pl.ds(c, 16)] = jnp.zeros((16,), F32)
            return carry

        lax.fori_loop(0, 1024, zb, 0)

        def zs(k, carry):
            c = k * NS + sid

            @pl.when(c < nch)
            def _():
                pltpu.sync_copy(mbuf, acc.at[pl.ds(c * 128, 128)])

            return carry

        lax.fori_loop(0, nch // NS + 1, zs, 0)

        @pl.when(sid == NS - 1)
        def _():
            pltpu.sync_copy(mbuf.at[pl.ds(0, tail)],
                            acc.at[pl.ds(nch * 128, tail)])

        plsc.subcore_barrier()

        def step(t, carry):
            jl = t * NS + sid

            @pl.when(jl < half)
            def _():
                j = cid * half + jl
                pltpu.sync_copy(cph.at[j], idx.at[0])
                pltpu.sync_copy(mh.at[pl.ds(j * 128, 128)], mbuf)
                pltpu.sync_copy(mbuf, acc.at[idx.at[0]], add=True)

            return carry

        lax.fori_loop(0, half // NS + 1, step, 0)
        plsc.subcore_barrier()

        def co(k, carry):
            c = k * NS + sid

            @pl.when(c < nch)
            def _():
                r0 = c * 128
                pltpu.sync_copy(acc.at[pl.ds(r0, 128)], mbuf)
                pltpu.sync_copy(mbuf, out.at[cid, pl.ds(r0, 128)])

            return carry

        lax.fori_loop(0, nch // NS + 1, co, 0)

        @pl.when(sid == NS - 1)
        def _():
            r0 = nch * 128
            pltpu.sync_copy(acc.at[pl.ds(r0, tail)], mbuf.at[pl.ds(0, tail)])
            pltpu.sync_copy(mbuf.at[pl.ds(0, tail)], out.at[cid, pl.ds(r0, tail)])

    return pl.kernel(
        body,
        out_type=jax.ShapeDtypeStruct((NC, N_NODES, H), F32),
        mesh=_mesh(),
        scratch_types=[
            pltpu.VMEM((1, 128), jnp.int32),
            pltpu.VMEM((128, H), F32),
            pltpu.VMEM_SHARED((N_NODES, H), F32),
            pltpu.SemaphoreType.DMA,
        ],
    )


def _sc_scatter(m, colp):
    return _make_sc_scatter()(m, colp)


def _node_enc_body(x, W0, b0, W1, b1, W2, b2, W3, b3, g, be, ho):
    t = jnp.dot(x[...], W0[...], preferred_element_type=F32) + b0[...]
    ho[...] = _mlp_tail(t, W1[...], b1[...], W2[...], b2[...], W3[...], b3[...],
                        g[...], be[...])


def _tc_node_enc(x, w):
    grid = N_NODES // MN
    specs = [pl.BlockSpec((MN, 4), lambda i: (i, 0))]
    specs += [_full(a.shape) for a in w]
    return pl.pallas_call(
        _node_enc_body,
        grid=(grid,),
        in_specs=specs,
        out_specs=pl.BlockSpec((MN, H), lambda i: (i, 0)),
        out_shape=jax.ShapeDtypeStruct((N_NODES, H), F32),
    )(x, *w)


def _edge_enc_body(pr, pc, W0, b0, W1, b1, W2, b2, W3, b3, g, be, eo):
    d = pr[...] - pc[...]
    t = jnp.dot(d, W0[...], preferred_element_type=F32) + b0[...]
    eo[...] = _mlp_tail(t, W1[...], b1[...], W2[...], b2[...], W3[...], b3[...],
                        g[...], be[...])


def _tc_edge_enc(prow, pcol, w):
    grid = N_EDGES // ME
    specs = [pl.BlockSpec((ME, 16), lambda i: (i, 0)),
             pl.BlockSpec((ME, 16), lambda i: (i, 0))]
    specs += [_full(a.shape) for a in w]
    return pl.pallas_call(
        _edge_enc_body,
        grid=(grid,),
        in_specs=specs,
        out_specs=pl.BlockSpec((ME, H), lambda i: (i, 0)),
        out_shape=jax.ShapeDtypeStruct((N_EDGES, H), F32),
    )(prow, pcol, *w)


def _edge_block_body(hr, hc, e, pr, u,
                     W0, b0, W1, b1, W2, b2, W3, b3, g, be,
                     V0, c0, V1, c1, V2, c2, V3, c3, vg, vbe,
                     eo, mo):
    hr_ = hr[...]
    hc_ = hc[...]
    e_ = e[...]
    bef = pr[...][:, 3].astype(jnp.int32)
    oh = (bef[:, None] == lax.broadcasted_iota(jnp.int32, (ME, N_GRAPHS), 1)
          ).astype(F32)
    ube = jnp.dot(oh, u[...], preferred_element_type=F32)
    W0_ = W0[...]
    t = (jnp.dot(hr_, W0_[0:128], preferred_element_type=F32)
         + jnp.dot(hc_, W0_[128:256], preferred_element_type=F32)
         + jnp.dot(e_, W0_[256:384], preferred_element_type=F32)
         + jnp.dot(ube, W0_[384:512], preferred_element_type=F32)
         + b0[...])
    de = _mlp_tail(t, W1[...], b1[...], W2[...], b2[...], W3[...], b3[...],
                   g[...], be[...])
    e_new = e_ + de
    eo[...] = e_new
    V0_ = V0[...]
    s = (jnp.dot(hr_, V0_[0:128], preferred_element_type=F32)
         + jnp.dot(e_new, V0_[128:256], preferred_element_type=F32)
         + c0[...])
    mo[...] = _mlp_tail(s, V1[...], c1[...], V2[...], c2[...], V3[...], c3[...],
                        vg[...], vbe[...])


def _tc_edge_block(hr, hc, e, prow, u, we, wv):
    grid = N_EDGES // ME
    em = lambda i: (i, 0)
    specs = [pl.BlockSpec((ME, H), em), pl.BlockSpec((ME, H), em),
             pl.BlockSpec((ME, H), em), pl.BlockSpec((ME, 16), em),
             _full((N_GRAPHS, H))]
    specs += [_full(a.shape) for a in we]
    specs += [_full(a.shape) for a in wv]
    return pl.pallas_call(
        _edge_block_body,
        grid=(grid,),
        in_specs=specs,
        out_specs=[pl.BlockSpec((ME, H), em), pl.BlockSpec((ME, H), em)],
        out_shape=[jax.ShapeDtypeStruct((N_EDGES, H), F32),
                   jax.ShapeDtypeStruct((N_EDGES, H), F32)],
    )(hr, hc, e, prow, u, *we, *wv)


def _node_block_body(h, a0, a1, T, u,
                     U0, d0, U1, d1, U2, d2, U3, d3, ug, ube,
                     Q0, q0, Q1, q1, Q2, q2, Q3, q3, qg, qbe,
                     ho, uo, gacc):
    i = pl.program_id(0)
    h_ = h[...]
    agg = a0[...] + a1[...]
    bf = T[...][:, 3].astype(jnp.int32)
    ohb = (bf[:, None] == lax.broadcasted_iota(jnp.int32, (MN, N_GRAPHS), 1)
           ).astype(F32)
    u_ = u[...]
    ub = jnp.dot(ohb, u_, preferred_element_type=F32)
    U0_ = U0[...]
    t = (jnp.dot(h_, U0_[0:128], preferred_element_type=F32)
         + jnp.dot(agg, U0_[128:256], preferred_element_type=F32)
         + jnp.dot(ub, U0_[256:384], preferred_element_type=F32)
         + d0[...])
    hn = h_ + _mlp_tail(t, U1[...], d1[...], U2[...], d2[...], U3[...], d3[...],
                        ug[...], ube[...])
    ho[...] = hn
    part = lax.dot_general(ohb, hn, (((0,), (0,)), ((), ())),
                           preferred_element_type=F32)

    @pl.when(i == 0)
    def _():
        gacc[...] = part

    @pl.when(i > 0)
    def _():
        gacc[...] = gacc[...] + part

    @pl.when(i == pl.num_programs(0) - 1)
    def _():
        Q0_ = Q0[...]
        tq = (jnp.dot(u_, Q0_[0:128], preferred_element_type=F32)
              + jnp.dot(gacc[...], Q0_[128:256], preferred_element_type=F32)
              + q0[...])
        uo[...] = u_ + _mlp_tail(tq, Q1[...], q1[...], Q2[...], q2[...],
                                 Q3[...], q3[...], qg[...], qbe[...])


def _tc_node_block(h, a0, a1, T, u, wu, wq):
    grid = N_NODES // MN
    em = lambda i: (i, 0)
    specs = [pl.BlockSpec((MN, H), em), pl.BlockSpec((MN, H), em),
             pl.BlockSpec((MN, H), em), pl.BlockSpec((MN, 16), em),
             _full((N_GRAPHS, H))]
    specs += [_full(a.shape) for a in wu]
    specs += [_full(a.shape) for a in wq]
    return pl.pallas_call(
        _node_block_body,
        grid=(grid,),
        in_specs=specs,
        out_specs=[pl.BlockSpec((MN, H), em), _full((N_GRAPHS, H))],
        out_shape=[jax.ShapeDtypeStruct((N_NODES, H), F32),
                   jax.ShapeDtypeStruct((N_GRAPHS, H), F32)],
        scratch_shapes=[pltpu.VMEM((N_GRAPHS, H), F32)],
    )(h, a0, a1, T, u, *wu, *wq)


def _decoder_body(h, W0, b0, W1, b1, W2, b2, W3, b3, yo):
    t = jnp.dot(h[...], W0[...], preferred_element_type=F32) + b0[...]
    yo[...] = _mlp_tail(t, W1[...], b1[...], W2[...], b2[...], W3[...], b3[...])


def _tc_decoder(h, w):
    grid = N_NODES // MN
    specs = [pl.BlockSpec((MN, H), lambda i: (i, 0))]
    specs += [_full(a.shape) for a in w]
    return pl.pallas_call(
        _decoder_body,
        grid=(grid,),
        in_specs=specs,
        out_specs=pl.BlockSpec((MN, H), lambda i: (i, 0)),
        out_shape=jax.ShapeDtypeStruct((N_NODES, H), F32),
    )(h, *w)


def kernel(x, edge_index, batch, params):
    rowp = edge_index[0].astype(jnp.int32).reshape(EP, 128)
    colp = edge_index[1].astype(jnp.int32).reshape(EP, 128)
    batchf = batch.astype(F32)
    T = jnp.concatenate(
        [x[:, :3], batchf[:, None], jnp.zeros((N_NODES, 12), F32)], axis=1)

    w_nenc = _pad_mlp(params["node_enc"], 4, True)
    w_nenc[0] = jnp.zeros((4, H), F32).at[3, :].set(w_nenc[0][0])
    w_eenc = _pad_mlp(params["edge_enc"], 16, True)
    w_dec = _pad_mlp(params["decoder"], H, False)

    prow, pcol = _sc_gather(T, rowp, colp)
    h = _tc_node_enc(x, w_nenc)
    e = _tc_edge_enc(prow, pcol, w_eenc)
    u = jnp.zeros((N_GRAPHS, H), F32)

    for bp in params["blocks"]:
        we = _pad_mlp(bp["edge"], 4 * H, True)
        wv = _pad_mlp(bp["node1"], 2 * H, True)
        wu = _pad_mlp(bp["node2"], 3 * H, True)
        wq = _pad_mlp(bp["glob"], 2 * H, True)
        hr, hc = _sc_gather(h, rowp, colp)
        e, m = _tc_edge_block(hr, hc, e, prow, u, we, wv)
        parts = _sc_scatter(m, colp)
        h, u = _tc_node_block(h, parts[0], parts[1], T, u, wu, wq)

    y = _tc_decoder(h, w_dec)
    return y[:, :1]


# speedup vs baseline: 4.0959x; 1.1877x over previous
import functools

import jax
import jax.numpy as jnp
from jax import lax
from jax.experimental import pallas as pl
from jax.experimental.pallas import tpu as pltpu
from jax.experimental.pallas import tpu_sc as plsc

F32 = jnp.float32
N_NODES = 10000
N_EDGES = 320000
N_GRAPHS = 64
H = 128
EP = N_EDGES // 128
NC, NS, NW = 2, 16, 32
ME = 2000
MN = 2000

@functools.lru_cache(maxsize=None)
def _mesh():
    return plsc.VectorSubcoreMesh(core_axis_name="c", subcore_axis_name="s",
                                  num_cores=NC, num_subcores=NS)


def _pad2(a, r, c):
    return jnp.pad(a, ((0, r - a.shape[0]), (0, c - a.shape[1])))


def _pad_mlp(p, n_in_pad, ln):
    Ws, bs = p["W"], p["b"]
    W0 = _pad2(Ws[0], n_in_pad, H)
    W1 = _pad2(Ws[1], H, H)
    W2 = _pad2(Ws[2], H, H)
    W3 = _pad2(Ws[3], H, H)
    b0 = jnp.pad(bs[0], (0, H - bs[0].shape[0]))[None, :]
    b1 = jnp.pad(bs[1], (0, H - bs[1].shape[0]))[None, :]
    b2 = jnp.pad(bs[2], (0, H - bs[2].shape[0]))[None, :]
    b3 = jnp.pad(bs[3], (0, H - bs[3].shape[0]))[None, :]
    out = [W0, b0, W1, b1, W2, b2, W3, b3]
    if ln:
        out += [jnp.pad(p["g"], (0, H - p["g"].shape[0]))[None, :],
                jnp.pad(p["be"], (0, H - p["be"].shape[0]))[None, :]]
    return out


def _mlp_tail(t, W1, b1, W2, b2, W3, b3, g=None, be=None):
    t = jnp.maximum(t, 0.0)
    t = jnp.maximum(jnp.dot(t, W1, preferred_element_type=F32) + b1, 0.0)
    t = jnp.maximum(jnp.dot(t, W2, preferred_element_type=F32) + b2, 0.0)
    t = jnp.dot(t, W3, preferred_element_type=F32) + b3
    if g is not None:
        mu = jnp.mean(t, axis=-1, keepdims=True)
        var = jnp.mean((t - mu) ** 2, axis=-1, keepdims=True)
        t = (t - mu) * lax.rsqrt(var + 1e-5) * g + be
    return t


def _full(shape):
    return pl.BlockSpec(shape, lambda *_: tuple(0 for _ in shape))


@functools.lru_cache(maxsize=None)
def _make_sc_gather(D, ep):
    per = ep // NW

    def body(tab, rp, cp, hr, hc, idx, bufr, bufc, sem1, sem2):
        cid = lax.axis_index("c")
        sid = lax.axis_index("s")
        wid = sid * NC + cid

        def step(t, carry):
            j = t * NW + wid

            @pl.when(j < ep)
            def _():
                pltpu.sync_copy(rp.at[j], idx.at[0])
                pltpu.sync_copy(cp.at[j], idx.at[1])
                cr = pltpu.async_copy(tab.at[idx.at[0]], bufr, sem1)
                cc = pltpu.async_copy(tab.at[idx.at[1]], bufc, sem2)
                cr.wait()
                cc.wait()
                pltpu.sync_copy(bufr, hr.at[pl.ds(j * 128, 128)])
                pltpu.sync_copy(bufc, hc.at[pl.ds(j * 128, 128)])

            return carry

        lax.fori_loop(0, per + 1, step, 0)

    fs = jax.ShapeDtypeStruct((ep * 128, D), F32)
    return pl.kernel(
        body,
        out_type=[fs, fs],
        mesh=_mesh(),
        compiler_params=pltpu.CompilerParams(use_tc_tiling_on_sc=(D == 128)),
        scratch_types=[
            pltpu.VMEM((2, 128), jnp.int32),
            pltpu.VMEM((128, D), F32),
            pltpu.VMEM((128, D), F32),
            pltpu.SemaphoreType.DMA,
            pltpu.SemaphoreType.DMA,
        ],
    )


def _sc_gather(table, rowp, colp):
    return _make_sc_gather(table.shape[1], rowp.shape[0])(table, rowp, colp)


@functools.lru_cache(maxsize=None)
def _make_sc_scatter(ep):
    half = ep // NC
    nch = N_NODES // 128
    tail = N_NODES - nch * 128

    def body(mh, cph, out, idx, mbuf, acc, sem):
        cid = lax.axis_index("c")
        sid = lax.axis_index("s")

        def zb(i, carry):
            r = i // 8
            c = (i % 8) * 16
            mbuf[r, pl.ds(c, 16)] = jnp.zeros((16,), F32)
            return carry

        lax.fori_loop(0, 1024, zb, 0)

        def zs(k, carry):
            c = k * NS + sid

            @pl.when(c < nch)
            def _():
                pltpu.sync_copy(mbuf, acc.at[pl.ds(c * 128, 128)])

            return carry

        lax.fori_loop(0, nch // NS + 1, zs, 0)

        @pl.when(sid == NS - 1)
        def _():
            pltpu.sync_copy(mbuf.at[pl.ds(0, tail)],
                            acc.at[pl.ds(nch * 128, tail)])

        plsc.subcore_barrier()

        def step(t, carry):
            jl = t * NS + sid

            @pl.when(jl < half)
            def _():
                j = cid * half + jl
                pltpu.sync_copy(cph.at[j], idx.at[0])
                pltpu.sync_copy(mh.at[pl.ds(j * 128, 128)], mbuf)
                pltpu.sync_copy(mbuf, acc.at[idx.at[0]], add=True)

            return carry

        lax.fori_loop(0, half // NS + 1, step, 0)
        plsc.subcore_barrier()

        def co(k, carry):
            c = k * NS + sid

            @pl.when(c < nch)
            def _():
                r0 = c * 128
                pltpu.sync_copy(acc.at[pl.ds(r0, 128)], mbuf)
                pltpu.sync_copy(mbuf, out.at[cid, pl.ds(r0, 128)])

            return carry

        lax.fori_loop(0, nch // NS + 1, co, 0)

        @pl.when(sid == NS - 1)
        def _():
            r0 = nch * 128
            pltpu.sync_copy(acc.at[pl.ds(r0, tail)], mbuf.at[pl.ds(0, tail)])
            pltpu.sync_copy(mbuf.at[pl.ds(0, tail)], out.at[cid, pl.ds(r0, tail)])

    return pl.kernel(
        body,
        out_type=jax.ShapeDtypeStruct((NC, N_NODES, H), F32),
        mesh=_mesh(),
        scratch_types=[
            pltpu.VMEM((1, 128), jnp.int32),
            pltpu.VMEM((128, H), F32),
            pltpu.VMEM_SHARED((N_NODES, H), F32),
            pltpu.SemaphoreType.DMA,
        ],
    )


def _sc_scatter(m, colp):
    return _make_sc_scatter(colp.shape[0])(m, colp)


def _node_enc_body(x, W0, b0, W1, b1, W2, b2, W3, b3, g, be, ho):
    t = jnp.dot(x[...], W0[...], preferred_element_type=F32) + b0[...]
    ho[...] = _mlp_tail(t, W1[...], b1[...], W2[...], b2[...], W3[...], b3[...],
                        g[...], be[...])


def _tc_node_enc(x, w):
    grid = N_NODES // MN
    specs = [pl.BlockSpec((MN, 4), lambda i: (i, 0))]
    specs += [_full(a.shape) for a in w]
    return pl.pallas_call(
        _node_enc_body,
        grid=(grid,),
        in_specs=specs,
        out_specs=pl.BlockSpec((MN, H), lambda i: (i, 0)),
        out_shape=jax.ShapeDtypeStruct((N_NODES, H), F32),
    )(x, *w)


def _edge_enc_body(pr, pc, W0, b0, W1, b1, W2, b2, W3, b3, g, be, eo):
    d = pr[...] - pc[...]
    t = jnp.dot(d, W0[...], preferred_element_type=F32) + b0[...]
    eo[...] = _mlp_tail(t, W1[...], b1[...], W2[...], b2[...], W3[...], b3[...],
                        g[...], be[...])


def _tc_edge_enc(prow, pcol, w, off, ne):
    grid = ne // ME
    specs = [pl.BlockSpec((ME, 16), lambda i: (i + off, 0)),
             pl.BlockSpec((ME, 16), lambda i: (i + off, 0))]
    specs += [_full(a.shape) for a in w]
    return pl.pallas_call(
        _edge_enc_body,
        grid=(grid,),
        in_specs=specs,
        out_specs=pl.BlockSpec((ME, H), lambda i: (i, 0)),
        out_shape=jax.ShapeDtypeStruct((ne, H), F32),
    )(prow, pcol, *w)


def _edge_block_body(hr, hc, e, pr, u,
                     W0, b0, W1, b1, W2, b2, W3, b3, g, be,
                     V0, c0, V1, c1, V2, c2, V3, c3, vg, vbe,
                     eo, mo):
    hr_ = hr[...]
    hc_ = hc[...]
    e_ = e[...]
    bef = pr[...][:, 3].astype(jnp.int32)
    oh = (bef[:, None] == lax.broadcasted_iota(jnp.int32, (ME, N_GRAPHS), 1)
          ).astype(F32)
    ube = jnp.dot(oh, u[...], preferred_element_type=F32)
    W0_ = W0[...]
    t = (jnp.dot(hr_, W0_[0:128], preferred_element_type=F32)
         + jnp.dot(hc_, W0_[128:256], preferred_element_type=F32)
         + jnp.dot(e_, W0_[256:384], preferred_element_type=F32)
         + jnp.dot(ube, W0_[384:512], preferred_element_type=F32)
         + b0[...])
    de = _mlp_tail(t, W1[...], b1[...], W2[...], b2[...], W3[...], b3[...],
                   g[...], be[...])
    e_new = e_ + de
    eo[...] = e_new
    V0_ = V0[...]
    s = (jnp.dot(hr_, V0_[0:128], preferred_element_type=F32)
         + jnp.dot(e_new, V0_[128:256], preferred_element_type=F32)
         + c0[...])
    mo[...] = _mlp_tail(s, V1[...], c1[...], V2[...], c2[...], V3[...], c3[...],
                        vg[...], vbe[...])


def _tc_edge_block(hr, hc, e, prow, u, we, wv, off):
    ne = hr.shape[0]
    grid = ne // ME
    em = lambda i: (i, 0)
    specs = [pl.BlockSpec((ME, H), em), pl.BlockSpec((ME, H), em),
             pl.BlockSpec((ME, H), em),
             pl.BlockSpec((ME, 16), lambda i: (i + off, 0)),
             _full((N_GRAPHS, H))]
    specs += [_full(a.shape) for a in we]
    specs += [_full(a.shape) for a in wv]
    return pl.pallas_call(
        _edge_block_body,
        grid=(grid,),
        in_specs=specs,
        out_specs=[pl.BlockSpec((ME, H), em), pl.BlockSpec((ME, H), em)],
        out_shape=[jax.ShapeDtypeStruct((ne, H), F32),
                   jax.ShapeDtypeStruct((ne, H), F32)],
    )(hr, hc, e, prow, u, *we, *wv)


def _node_block_body(h, a0, a1, a2, a3, T, u,
                     U0, d0, U1, d1, U2, d2, U3, d3, ug, ube,
                     Q0, q0, Q1, q1, Q2, q2, Q3, q3, qg, qbe,
                     ho, uo, gacc):
    i = pl.program_id(0)
    h_ = h[...]
    agg = (a0[...] + a1[...]) + (a2[...] + a3[...])
    bf = T[...][:, 3].astype(jnp.int32)
    ohb = (bf[:, None] == lax.broadcasted_iota(jnp.int32, (MN, N_GRAPHS), 1)
           ).astype(F32)
    u_ = u[...]
    ub = jnp.dot(ohb, u_, preferred_element_type=F32)
    U0_ = U0[...]
    t = (jnp.dot(h_, U0_[0:128], preferred_element_type=F32)
         + jnp.dot(agg, U0_[128:256], preferred_element_type=F32)
         + jnp.dot(ub, U0_[256:384], preferred_element_type=F32)
         + d0[...])
    hn = h_ + _mlp_tail(t, U1[...], d1[...], U2[...], d2[...], U3[...], d3[...],
                        ug[...], ube[...])
    ho[...] = hn
    part = lax.dot_general(ohb, hn, (((0,), (0,)), ((), ())),
                           preferred_element_type=F32)

    @pl.when(i == 0)
    def _():
        gacc[...] = part

    @pl.when(i > 0)
    def _():
        gacc[...] = gacc[...] + part

    @pl.when(i == pl.num_programs(0) - 1)
    def _():
        Q0_ = Q0[...]
        tq = (jnp.dot(u_, Q0_[0:128], preferred_element_type=F32)
              + jnp.dot(gacc[...], Q0_[128:256], preferred_element_type=F32)
              + q0[...])
        uo[...] = u_ + _mlp_tail(tq, Q1[...], q1[...], Q2[...], q2[...],
                                 Q3[...], q3[...], qg[...], qbe[...])


def _tc_node_block(h, a0, a1, a2, a3, T, u, wu, wq):
    grid = N_NODES // MN
    em = lambda i: (i, 0)
    specs = [pl.BlockSpec((MN, H), em), pl.BlockSpec((MN, H), em),
             pl.BlockSpec((MN, H), em), pl.BlockSpec((MN, H), em),
             pl.BlockSpec((MN, H), em), pl.BlockSpec((MN, 16), em),
             _full((N_GRAPHS, H))]
    specs += [_full(a.shape) for a in wu]
    specs += [_full(a.shape) for a in wq]
    return pl.pallas_call(
        _node_block_body,
        grid=(grid,),
        in_specs=specs,
        out_specs=[pl.BlockSpec((MN, H), em), _full((N_GRAPHS, H))],
        out_shape=[jax.ShapeDtypeStruct((N_NODES, H), F32),
                   jax.ShapeDtypeStruct((N_GRAPHS, H), F32)],
        scratch_shapes=[pltpu.VMEM((N_GRAPHS, H), F32)],
    )(h, a0, a1, a2, a3, T, u, *wu, *wq)


def _decoder_body(h, W0, b0, W1, b1, W2, b2, W3, b3, yo):
    t = jnp.dot(h[...], W0[...], preferred_element_type=F32) + b0[...]
    yo[...] = _mlp_tail(t, W1[...], b1[...], W2[...], b2[...], W3[...], b3[...])


def _tc_decoder(h, w):
    grid = N_NODES // MN
    specs = [pl.BlockSpec((MN, H), lambda i: (i, 0))]
    specs += [_full(a.shape) for a in w]
    return pl.pallas_call(
        _decoder_body,
        grid=(grid,),
        in_specs=specs,
        out_specs=pl.BlockSpec((MN, H), lambda i: (i, 0)),
        out_shape=jax.ShapeDtypeStruct((N_NODES, H), F32),
    )(h, *w)


def kernel(x, edge_index, batch, params):
    rowp = edge_index[0].astype(jnp.int32).reshape(EP, 128)
    colp = edge_index[1].astype(jnp.int32).reshape(EP, 128)
    batchf = batch.astype(F32)
    T = jnp.concatenate(
        [x[:, :3], batchf[:, None], jnp.zeros((N_NODES, 12), F32)], axis=1)

    w_nenc = _pad_mlp(params["node_enc"], 4, True)
    w_nenc[0] = jnp.zeros((4, H), F32).at[3, :].set(w_nenc[0][0])
    w_eenc = _pad_mlp(params["edge_enc"], 16, True)
    w_dec = _pad_mlp(params["decoder"], H, False)

    eph = EP // 2
    eh = eph * 128
    rpA, rpB = rowp[:eph], rowp[eph:]
    cpA, cpB = colp[:eph], colp[eph:]
    offB = eh // ME

    prow, pcol = _sc_gather(T, rowp, colp)
    h = _tc_node_enc(x, w_nenc)
    eA = _tc_edge_enc(prow, pcol, w_eenc, 0, eh)
    eB = _tc_edge_enc(prow, pcol, w_eenc, offB, eh)
    u = jnp.zeros((N_GRAPHS, H), F32)

    for bp in params["blocks"]:
        we = _pad_mlp(bp["edge"], 4 * H, True)
        wv = _pad_mlp(bp["node1"], 2 * H, True)
        wu = _pad_mlp(bp["node2"], 3 * H, True)
        wq = _pad_mlp(bp["glob"], 2 * H, True)
        hrA, hcA = _sc_gather(h, rpA, cpA)
        hrB, hcB = _sc_gather(h, rpB, cpB)
        eA, mA = _tc_edge_block(hrA, hcA, eA, prow, u, we, wv, 0)
        pA = _sc_scatter(mA, cpA)
        eB, mB = _tc_edge_block(hrB, hcB, eB, prow, u, we, wv, offB)
        pB = _sc_scatter(mB, cpB)
        h, u = _tc_node_block(h, pA[0], pA[1], pB[0], pB[1], T, u, wu, wq)

    y = _tc_decoder(h, w_dec)
    return y[:, :1]
